# Initial kernel scaffold; baseline (speedup 1.0000x reference)
import jax
import jax.numpy as jnp
from jax.experimental import pallas as pl


def kernel(x, edge_index, W0, b0, W1, b1, W2, b2):
    raise NotImplementedError("write your pallas kernel here")


# speedup vs baseline: 5.0652x; 5.0652x over previous
import functools

import jax
import jax.numpy as jnp
from jax import lax
from jax.experimental import pallas as pl
from jax.experimental.pallas import tpu as pltpu
from jax.experimental.pallas import tpu_sc as plsc

N = 10000
NPAD = 10240
E = 320000
D = 128

NC = 2
NS = 16
NW = NC * NS
EPW = E // NW
C = 80
ITERS = EPW // C
RPS = NPAD // NS
ZR = 64

_mesh = plsc.VectorSubcoreMesh(
    core_axis_name="c", subcore_axis_name="s", num_cores=NC, num_subcores=NS
)


def _zero_fill(buf, rows, cols):
    zv = jnp.zeros((16,), jnp.float32)

    @pl.loop(0, rows)
    def _(r):
        @pl.loop(0, cols, step=16)
        def _(j):
            buf.at[r, pl.ds(j, 16)][...] = zv


def _sc_deg(row, width=16, lin=True):
    @functools.partial(
        pl.kernel,
        out_type=jax.ShapeDtypeStruct((NC, NPAD, width), jnp.float32),
        mesh=_mesh,
        scratch_types=[
            pltpu.VMEM((C,), jnp.int32),
            pltpu.VMEM((C, width), jnp.float32),
            pltpu.VMEM((ZR, width), jnp.float32),
            pltpu.VMEM_SHARED((NPAD, width), jnp.float32),
            pltpu.SemaphoreType.DMA,
        ],
        compiler_params=pltpu.CompilerParams(use_tc_tiling_on_sc=not lin),
    )
    def k(row_hbm, pdeg_hbm, idx_r, ones_v, zdeg, deg_sh, sem):
        c = lax.axis_index("c")
        s = lax.axis_index("s")
        wid = s * NC + c

        _zero_fill(zdeg, ZR, width)
        ov = jnp.ones((16,), jnp.float32)

        @pl.loop(0, C)
        def _(r):
            @pl.loop(0, width, step=16)
            def _(j):
                ones_v.at[r, pl.ds(j, 16)][...] = ov

        rs = s * RPS

        @pl.loop(0, RPS // ZR)
        def _(t):
            pltpu.sync_copy(zdeg, deg_sh.at[pl.ds(rs + t * ZR, ZR)])

        plsc.subcore_barrier()

        base = wid * EPW

        @pl.loop(0, ITERS)
        def _(i):
            off = base + i * C
            pltpu.sync_copy(row_hbm.at[pl.ds(off, C)], idx_r)
            pltpu.sync_copy(ones_v, deg_sh.at[idx_r], add=True)

        plsc.subcore_barrier()
        pltpu.sync_copy(deg_sh.at[pl.ds(rs, RPS)],
                        pdeg_hbm.at[c, pl.ds(rs, RPS)])

    return k(row)


def _sc_aggr(x, row, col):
    @functools.partial(
        pl.kernel,
        out_type=jax.ShapeDtypeStruct((NC, NPAD, D), jnp.float32),
        mesh=_mesh,
        scratch_types=[
            pltpu.VMEM((C,), jnp.int32),
            pltpu.VMEM((C,), jnp.int32),
            pltpu.VMEM((C, D), jnp.float32),
            pltpu.VMEM((ZR, D), jnp.float32),
            pltpu.VMEM_SHARED((NPAD, D), jnp.float32),
            pltpu.SemaphoreType.DMA,
        ],
    )
    def k(x_hbm, row_hbm, col_hbm, paggr_hbm, idx_r, idx_c, rows_v, zrow,
          aggr_sh, sem):
        c = lax.axis_index("c")
        s = lax.axis_index("s")
        wid = s * NC + c

        _zero_fill(zrow, ZR, D)
        rs = s * RPS

        @pl.loop(0, RPS // ZR)
        def _(t):
            pltpu.sync_copy(zrow, aggr_sh.at[pl.ds(rs + t * ZR, ZR)])

        plsc.subcore_barrier()

        base = wid * EPW

        @pl.loop(0, ITERS)
        def _(i):
            off = base + i * C
            pltpu.sync_copy(row_hbm.at[pl.ds(off, C)], idx_r)
            pltpu.sync_copy(col_hbm.at[pl.ds(off, C)], idx_c)
            pltpu.async_copy(x_hbm.at[idx_r], rows_v, sem).wait()
            pltpu.sync_copy(rows_v, aggr_sh.at[idx_c], add=True)

        plsc.subcore_barrier()
        pltpu.sync_copy(aggr_sh.at[pl.ds(rs, RPS)],
                        paggr_hbm.at[c, pl.ds(rs, RPS)])

    return k(x, row, col)


BR = 2048


def _combine_body(relu, p_ref, d_ref, x_ref, w_ref, b_ref, o_ref):
    d = d_ref[0, :, 0:1] + d_ref[1, :, 0:1]
    inv = 1.0 / jnp.maximum(d, 1.0)
    a = (p_ref[0] + p_ref[1]) * inv + x_ref[...]
    y = jnp.dot(a, w_ref[...], preferred_element_type=jnp.float32) + b_ref[...]
    o_ref[...] = jnp.maximum(y, 0.0) if relu else y


def _combine(p, degp, x, w, b, relu):
    return pl.pallas_call(
        functools.partial(_combine_body, relu),
        grid=(NPAD // BR,),
        in_specs=[
            pl.BlockSpec((NC, BR, D), lambda i: (0, i, 0)),
            pl.BlockSpec((NC, BR, 16), lambda i: (0, i, 0)),
            pl.BlockSpec((BR, D), lambda i: (i, 0)),
            pl.BlockSpec((D, D), lambda i: (0, 0)),
            pl.BlockSpec((1, D), lambda i: (0, 0)),
        ],
        out_specs=pl.BlockSpec((BR, D), lambda i: (i, 0)),
        out_shape=jax.ShapeDtypeStruct((NPAD, D), jnp.float32),
    )(p, degp, x, w, b.reshape(1, D))


def kernel(x, edge_index, W0, b0, W1, b1, W2, b2):
    ei = edge_index.astype(jnp.int32)
    row = ei[0]
    col = ei[1]
    xp = jnp.pad(x, ((0, NPAD - N), (0, 0)))

    degp = _sc_deg(row)
    p1 = _sc_aggr(xp, row, col)
    h1 = _combine(p1, degp, xp, W0, b0, relu=True)
    p2 = _sc_aggr(h1, row, col)
    h2 = _combine(p2, degp, h1, W1, b1, relu=True)
    p3 = _sc_aggr(h2, row, col)
    out = _combine(p3, degp, h2, W2, b2, relu=False)
    return out[:N]


# speedup vs baseline: 8.0045x; 1.5803x over previous
import functools

import jax
import jax.numpy as jnp
from jax import lax
from jax.experimental import pallas as pl
from jax.experimental.pallas import tpu as pltpu
from jax.experimental.pallas import tpu_sc as plsc

N = 10000
NPAD = 10240
E = 320000
D = 128

NC = 2
NS = 16
NW = NC * NS
EPW = E // NW
C = 80
CD = 1000
ITERS = EPW // C
RPS = NPAD // NS
ZR = 64

_mesh = plsc.VectorSubcoreMesh(
    core_axis_name="c", subcore_axis_name="s", num_cores=NC, num_subcores=NS
)


def _zero_fill(buf, rows, cols):
    zv = jnp.zeros((16,), jnp.float32)

    @pl.loop(0, rows)
    def _(r):
        @pl.loop(0, cols, step=16)
        def _(j):
            buf.at[r, pl.ds(j, 16)][...] = zv


def _sc_deg(row, width=16, lin=True):
    @functools.partial(
        pl.kernel,
        out_type=jax.ShapeDtypeStruct((NC, NPAD, width), jnp.float32),
        mesh=_mesh,
        scratch_types=[
            pltpu.VMEM((CD,), jnp.int32),
            pltpu.VMEM((CD, width), jnp.float32),
            pltpu.VMEM((ZR, width), jnp.float32),
            pltpu.VMEM_SHARED((NPAD, width), jnp.float32),
            pltpu.SemaphoreType.DMA,
        ],
        compiler_params=pltpu.CompilerParams(use_tc_tiling_on_sc=not lin),
    )
    def k(row_hbm, pdeg_hbm, idx_r, ones_v, zdeg, deg_sh, sem):
        c = lax.axis_index("c")
        s = lax.axis_index("s")
        wid = s * NC + c

        _zero_fill(zdeg, ZR, width)
        ov = jnp.ones((16,), jnp.float32)

        @pl.loop(0, CD)
        def _(r):
            @pl.loop(0, width, step=16)
            def _(j):
                ones_v.at[r, pl.ds(j, 16)][...] = ov

        rs = s * RPS

        @pl.loop(0, RPS // ZR)
        def _(t):
            pltpu.sync_copy(zdeg, deg_sh.at[pl.ds(rs + t * ZR, ZR)])

        plsc.subcore_barrier()

        base = wid * EPW

        @pl.loop(0, EPW // CD)
        def _(i):
            off = base + i * CD
            pltpu.sync_copy(row_hbm.at[pl.ds(off, CD)], idx_r)
            pltpu.sync_copy(ones_v, deg_sh.at[idx_r], add=True)

        plsc.subcore_barrier()
        pltpu.sync_copy(deg_sh.at[pl.ds(rs, RPS)],
                        pdeg_hbm.at[c, pl.ds(rs, RPS)])

    return k(row)


def _sc_aggr(x, row, col):
    @functools.partial(
        pl.kernel,
        out_type=jax.ShapeDtypeStruct((NC, NPAD, D), jnp.float32),
        mesh=_mesh,
        scratch_types=[
            pltpu.VMEM((C,), jnp.int32),
            pltpu.VMEM((C,), jnp.int32),
            pltpu.VMEM((C,), jnp.int32),
            pltpu.VMEM((C,), jnp.int32),
            pltpu.VMEM((C, D), jnp.float32),
            pltpu.VMEM((C, D), jnp.float32),
            pltpu.VMEM((ZR, D), jnp.float32),
            pltpu.VMEM_SHARED((NPAD, D), jnp.float32),
            pltpu.SemaphoreType.DMA,
            pltpu.SemaphoreType.DMA,
            pltpu.SemaphoreType.DMA,
            pltpu.SemaphoreType.DMA,
        ],
    )
    def k(x_hbm, row_hbm, col_hbm, paggr_hbm, idx_r0, idx_c0, idx_r1, idx_c1,
          rows0, rows1, zrow, aggr_sh, semg0, semg1, sems0, sems1):
        c = lax.axis_index("c")
        s = lax.axis_index("s")
        wid = s * NC + c

        _zero_fill(zrow, ZR, D)
        rs = s * RPS

        @pl.loop(0, RPS // ZR)
        def _(t):
            pltpu.sync_copy(zrow, aggr_sh.at[pl.ds(rs + t * ZR, ZR)])

        plsc.subcore_barrier()

        base = wid * EPW

        def load_idx(ir, ic, off):
            pltpu.sync_copy(row_hbm.at[pl.ds(off, C)], ir)
            pltpu.sync_copy(col_hbm.at[pl.ds(off, C)], ic)

        load_idx(idx_r0, idx_c0, base)
        pltpu.async_copy(x_hbm.at[idx_r0], rows0, semg0)
        load_idx(idx_r1, idx_c1, base + C)
        pltpu.async_copy(x_hbm.at[idx_r1], rows1, semg1)

        @pl.loop(0, ITERS // 2 - 1)
        def _(g):
            off = base + (2 * g + 2) * C
            pltpu.make_async_copy(x_hbm.at[idx_r0], rows0, semg0).wait()
            pltpu.async_copy(rows0, aggr_sh.at[idx_c0], sems0, add=True)
            pltpu.make_async_copy(x_hbm.at[idx_r1], rows1, semg1).wait()
            pltpu.async_copy(rows1, aggr_sh.at[idx_c1], sems1, add=True)
            pltpu.make_async_copy(rows0, aggr_sh.at[idx_c0], sems0).wait()
            load_idx(idx_r0, idx_c0, off)
            pltpu.async_copy(x_hbm.at[idx_r0], rows0, semg0)
            pltpu.make_async_copy(rows1, aggr_sh.at[idx_c1], sems1).wait()
            load_idx(idx_r1, idx_c1, off + C)
            pltpu.async_copy(x_hbm.at[idx_r1], rows1, semg1)

        pltpu.make_async_copy(x_hbm.at[idx_r0], rows0, semg0).wait()
        pltpu.async_copy(rows0, aggr_sh.at[idx_c0], sems0, add=True)
        pltpu.make_async_copy(x_hbm.at[idx_r1], rows1, semg1).wait()
        pltpu.async_copy(rows1, aggr_sh.at[idx_c1], sems1, add=True)
        pltpu.make_async_copy(rows0, aggr_sh.at[idx_c0], sems0).wait()
        pltpu.make_async_copy(rows1, aggr_sh.at[idx_c1], sems1).wait()

        if ITERS % 2 == 1:
            off = base + (ITERS - 1) * C
            load_idx(idx_r0, idx_c0, off)
            pltpu.async_copy(x_hbm.at[idx_r0], rows0, semg0).wait()
            pltpu.sync_copy(rows0, aggr_sh.at[idx_c0], add=True)

        plsc.subcore_barrier()
        pltpu.sync_copy(aggr_sh.at[pl.ds(rs, RPS)],
                        paggr_hbm.at[c, pl.ds(rs, RPS)])

    return k(x, row, col)


BR = 2048


def _combine_body(relu, p_ref, d_ref, x_ref, w_ref, b_ref, o_ref):
    d = d_ref[0, :, 0:1] + d_ref[1, :, 0:1]
    inv = 1.0 / jnp.maximum(d, 1.0)
    a = (p_ref[0] + p_ref[1]) * inv + x_ref[...]
    y = jnp.dot(a, w_ref[...], preferred_element_type=jnp.float32) + b_ref[...]
    o_ref[...] = jnp.maximum(y, 0.0) if relu else y


def _combine(p, degp, x, w, b, relu):
    return pl.pallas_call(
        functools.partial(_combine_body, relu),
        grid=(NPAD // BR,),
        in_specs=[
            pl.BlockSpec((NC, BR, D), lambda i: (0, i, 0)),
            pl.BlockSpec((NC, BR, 16), lambda i: (0, i, 0)),
            pl.BlockSpec((BR, D), lambda i: (i, 0)),
            pl.BlockSpec((D, D), lambda i: (0, 0)),
            pl.BlockSpec((1, D), lambda i: (0, 0)),
        ],
        out_specs=pl.BlockSpec((BR, D), lambda i: (i, 0)),
        out_shape=jax.ShapeDtypeStruct((NPAD, D), jnp.float32),
    )(p, degp, x, w, b.reshape(1, D))


def kernel(x, edge_index, W0, b0, W1, b1, W2, b2):
    ei = edge_index.astype(jnp.int32)
    row = ei[0]
    col = ei[1]
    xp = jnp.pad(x, ((0, NPAD - N), (0, 0)))

    degp = _sc_deg(row)
    p1 = _sc_aggr(xp, row, col)
    h1 = _combine(p1, degp, xp, W0, b0, relu=True)
    p2 = _sc_aggr(h1, row, col)
    h2 = _combine(p2, degp, h1, W1, b1, relu=True)
    p3 = _sc_aggr(h2, row, col)
    out = _combine(p3, degp, h2, W2, b2, relu=False)
    return out[:N]


# speedup vs baseline: 9.8089x; 1.2254x over previous
import functools

import jax
import jax.numpy as jnp
from jax import lax
from jax.experimental import pallas as pl
from jax.experimental.pallas import tpu as pltpu
from jax.experimental.pallas import tpu_sc as plsc

N = 10000
NPAD = 10240
E = 320000
D = 128

NC = 2
NS = 16
NW = NC * NS
EPW = E // NW
C = 80
CD = 1000
ITERS = EPW // C
RPS = NPAD // NS
ZR = 32

_mesh = plsc.VectorSubcoreMesh(
    core_axis_name="c", subcore_axis_name="s", num_cores=NC, num_subcores=NS
)


def _zero_fill(buf, rows, cols):
    zv = jnp.zeros((16,), jnp.float32)

    @pl.loop(0, rows)
    def _(r):
        @pl.loop(0, cols, step=16)
        def _(j):
            buf.at[r, pl.ds(j, 16)][...] = zv


def _sc_deg(row, width=16, lin=True):
    @functools.partial(
        pl.kernel,
        out_type=jax.ShapeDtypeStruct((NC, NPAD, width), jnp.float32),
        mesh=_mesh,
        scratch_types=[
            pltpu.VMEM((CD,), jnp.int32),
            pltpu.VMEM((CD, width), jnp.float32),
            pltpu.VMEM((ZR, width), jnp.float32),
            pltpu.VMEM_SHARED((NPAD, width), jnp.float32),
            pltpu.SemaphoreType.DMA,
        ],
        compiler_params=pltpu.CompilerParams(use_tc_tiling_on_sc=not lin),
    )
    def k(row_hbm, pdeg_hbm, idx_r, ones_v, zdeg, deg_sh, sem):
        c = lax.axis_index("c")
        s = lax.axis_index("s")
        wid = s * NC + c

        _zero_fill(zdeg, ZR, width)
        ov = jnp.ones((16,), jnp.float32)

        @pl.loop(0, CD)
        def _(r):
            @pl.loop(0, width, step=16)
            def _(j):
                ones_v.at[r, pl.ds(j, 16)][...] = ov

        rs = s * RPS

        @pl.loop(0, RPS // ZR)
        def _(t):
            pltpu.sync_copy(zdeg, deg_sh.at[pl.ds(rs + t * ZR, ZR)])

        plsc.subcore_barrier()

        base = wid * EPW

        @pl.loop(0, EPW // CD)
        def _(i):
            off = base + i * CD
            pltpu.sync_copy(row_hbm.at[pl.ds(off, CD)], idx_r)
            pltpu.sync_copy(ones_v, deg_sh.at[idx_r], add=True)

        plsc.subcore_barrier()
        pltpu.sync_copy(deg_sh.at[pl.ds(rs, RPS)],
                        pdeg_hbm.at[c, pl.ds(rs, RPS)])

    return k(row)


ICH = EPW // C


def _sc_aggr(x, row3, col3):
    @functools.partial(
        pl.kernel,
        out_type=jax.ShapeDtypeStruct((NC, NPAD, D), jnp.float32),
        mesh=_mesh,
        scratch_types=[
            pltpu.VMEM((ICH, C), jnp.int32),
            pltpu.VMEM((ICH, C), jnp.int32),
            pltpu.VMEM((C, D), jnp.float32),
            pltpu.VMEM((C, D), jnp.float32),
            pltpu.VMEM((ZR, D), jnp.float32),
            pltpu.VMEM_SHARED((NPAD, D), jnp.float32),
            pltpu.SemaphoreType.DMA,
            pltpu.SemaphoreType.DMA,
            pltpu.SemaphoreType.DMA,
            pltpu.SemaphoreType.DMA,
        ],
        compiler_params=pltpu.CompilerParams(use_tc_tiling_on_sc=False),
    )
    def k(x_hbm, row_hbm, col_hbm, paggr_hbm, idx_r, idx_c,
          rows0, rows1, zrow, aggr_sh, semg0, semg1, sems0, sems1):
        c = lax.axis_index("c")
        s = lax.axis_index("s")
        wid = s * NC + c

        _zero_fill(zrow, ZR, D)
        rs = s * RPS

        @pl.loop(0, RPS // ZR)
        def _(t):
            pltpu.sync_copy(zrow, aggr_sh.at[pl.ds(rs + t * ZR, ZR)])

        pltpu.sync_copy(row_hbm.at[wid], idx_r)
        pltpu.sync_copy(col_hbm.at[wid], idx_c)

        plsc.subcore_barrier()

        pltpu.async_copy(x_hbm.at[idx_r.at[0]], rows0, semg0)
        pltpu.async_copy(x_hbm.at[idx_r.at[1]], rows1, semg1)

        @pl.loop(0, ITERS // 2 - 1)
        def _(g):
            j = 2 * g
            pltpu.make_async_copy(x_hbm.at[idx_r.at[j]], rows0, semg0).wait()
            pltpu.async_copy(rows0, aggr_sh.at[idx_c.at[j]], sems0, add=True)
            pltpu.make_async_copy(x_hbm.at[idx_r.at[j + 1]], rows1, semg1).wait()
            pltpu.async_copy(rows1, aggr_sh.at[idx_c.at[j + 1]], sems1, add=True)
            pltpu.make_async_copy(rows0, aggr_sh.at[idx_c.at[j]], sems0).wait()
            pltpu.async_copy(x_hbm.at[idx_r.at[j + 2]], rows0, semg0)
            pltpu.make_async_copy(rows1, aggr_sh.at[idx_c.at[j + 1]], sems1).wait()
            pltpu.async_copy(x_hbm.at[idx_r.at[j + 3]], rows1, semg1)

        jl = ITERS - 2 if ITERS % 2 == 0 else ITERS - 3
        pltpu.make_async_copy(x_hbm.at[idx_r.at[jl]], rows0, semg0).wait()
        pltpu.async_copy(rows0, aggr_sh.at[idx_c.at[jl]], sems0, add=True)
        pltpu.make_async_copy(x_hbm.at[idx_r.at[jl + 1]], rows1, semg1).wait()
        pltpu.async_copy(rows1, aggr_sh.at[idx_c.at[jl + 1]], sems1, add=True)
        pltpu.make_async_copy(rows0, aggr_sh.at[idx_c.at[jl]], sems0).wait()
        pltpu.make_async_copy(rows1, aggr_sh.at[idx_c.at[jl + 1]], sems1).wait()

        if ITERS % 2 == 1:
            pltpu.async_copy(x_hbm.at[idx_r.at[ITERS - 1]], rows0, semg0).wait()
            pltpu.sync_copy(rows0, aggr_sh.at[idx_c.at[ITERS - 1]], add=True)

        plsc.subcore_barrier()
        pltpu.sync_copy(aggr_sh.at[pl.ds(rs, RPS)],
                        paggr_hbm.at[c, pl.ds(rs, RPS)])

    return k(x, row3, col3)


BR = 2048


def _combine_body(relu, p_ref, d_ref, x_ref, w_ref, b_ref, o_ref):
    d = d_ref[0, :, 0:1] + d_ref[1, :, 0:1]
    inv = 1.0 / jnp.maximum(d, 1.0)
    a = (p_ref[0] + p_ref[1]) * inv + x_ref[...]
    y = jnp.dot(a, w_ref[...], preferred_element_type=jnp.float32) + b_ref[...]
    o_ref[...] = jnp.maximum(y, 0.0) if relu else y


def _combine(p, degp, x, w, b, relu):
    return pl.pallas_call(
        functools.partial(_combine_body, relu),
        grid=(NPAD // BR,),
        in_specs=[
            pl.BlockSpec((NC, BR, D), lambda i: (0, i, 0)),
            pl.BlockSpec((NC, BR, 16), lambda i: (0, i, 0)),
            pl.BlockSpec((BR, D), lambda i: (i, 0)),
            pl.BlockSpec((D, D), lambda i: (0, 0)),
            pl.BlockSpec((1, D), lambda i: (0, 0)),
        ],
        out_specs=pl.BlockSpec((BR, D), lambda i: (i, 0)),
        out_shape=jax.ShapeDtypeStruct((NPAD, D), jnp.float32),
    )(p, degp, x, w, b.reshape(1, D))


def kernel(x, edge_index, W0, b0, W1, b1, W2, b2):
    ei = edge_index.astype(jnp.int32)
    row = ei[0]
    col = ei[1]
    xp = jnp.pad(x, ((0, NPAD - N), (0, 0)))

    degp = _sc_deg(row)
    row3 = row.reshape(NW, ICH, C)
    col3 = col.reshape(NW, ICH, C)
    p1 = _sc_aggr(xp, row3, col3)
    h1 = _combine(p1, degp, xp, W0, b0, relu=True)
    p2 = _sc_aggr(h1, row3, col3)
    h2 = _combine(p2, degp, h1, W1, b1, relu=True)
    p3 = _sc_aggr(h2, row3, col3)
    out = _combine(p3, degp, h2, W2, b2, relu=False)
    return out[:N]


# speedup vs baseline: 11.5900x; 1.1816x over previous
import functools

import jax
import jax.numpy as jnp
from jax import lax
from jax.experimental import pallas as pl
from jax.experimental.pallas import tpu as pltpu
from jax.experimental.pallas import tpu_sc as plsc

N = 10000
NPAD = 10240
E = 320000
D = 128

NC = 2
NS = 16
NW = NC * NS
EPW = E // NW
C = 80
CD = 1000
ITERS = EPW // C
RPS = NPAD // NS
ZR = 32

_mesh = plsc.VectorSubcoreMesh(
    core_axis_name="c", subcore_axis_name="s", num_cores=NC, num_subcores=NS
)


def _zero_fill(buf, rows, cols):
    zv = jnp.zeros((16,), jnp.float32)

    @pl.loop(0, rows)
    def _(r):
        @pl.loop(0, cols, step=16)
        def _(j):
            buf.at[r, pl.ds(j, 16)][...] = zv


def _sc_deg(row, width=16, lin=True):
    @functools.partial(
        pl.kernel,
        out_type=jax.ShapeDtypeStruct((NC, NPAD, width), jnp.float32),
        mesh=_mesh,
        scratch_types=[
            pltpu.VMEM((CD,), jnp.int32),
            pltpu.VMEM((CD, width), jnp.float32),
            pltpu.VMEM((ZR, width), jnp.float32),
            pltpu.VMEM_SHARED((NPAD, width), jnp.float32),
            pltpu.SemaphoreType.DMA,
        ],
        compiler_params=pltpu.CompilerParams(use_tc_tiling_on_sc=not lin),
    )
    def k(row_hbm, pdeg_hbm, idx_r, ones_v, zdeg, deg_sh, sem):
        c = lax.axis_index("c")
        s = lax.axis_index("s")
        wid = s * NC + c

        _zero_fill(zdeg, ZR, width)
        ov = jnp.ones((16,), jnp.float32)

        @pl.loop(0, CD)
        def _(r):
            @pl.loop(0, width, step=16)
            def _(j):
                ones_v.at[r, pl.ds(j, 16)][...] = ov

        rs = s * RPS

        @pl.loop(0, RPS // ZR)
        def _(t):
            pltpu.sync_copy(zdeg, deg_sh.at[pl.ds(rs + t * ZR, ZR)])

        plsc.subcore_barrier()

        base = wid * EPW

        @pl.loop(0, EPW // CD)
        def _(i):
            off = base + i * CD
            pltpu.sync_copy(row_hbm.at[pl.ds(off, CD)], idx_r)
            pltpu.sync_copy(ones_v, deg_sh.at[idx_r], add=True)

        plsc.subcore_barrier()
        pltpu.sync_copy(deg_sh.at[pl.ds(rs, RPS)],
                        pdeg_hbm.at[c, pl.ds(rs, RPS)])

    return k(row)


ICH = EPW // C
NB = 3
IB = 63
PHASES = ((0, 62), (62, 63))


def _sc_aggr(x, row3, col3):
    @functools.partial(
        pl.kernel,
        out_type=jax.ShapeDtypeStruct((NC, NPAD, D), jnp.float32),
        mesh=_mesh,
        scratch_types=[
            pltpu.VMEM((IB, C), jnp.int32),
            pltpu.VMEM((IB, C), jnp.int32),
        ] + [pltpu.VMEM((C, D), jnp.float32)] * NB + [
            pltpu.VMEM_SHARED((NPAD, D), jnp.float32),
        ] + [pltpu.SemaphoreType.DMA] * (2 * NB),
        compiler_params=pltpu.CompilerParams(use_tc_tiling_on_sc=False),
    )
    def k(x_hbm, row_hbm, col_hbm, paggr_hbm, idx_r, idx_c, *rest):
        rows = rest[:NB]
        aggr_sh = rest[NB]
        semg = rest[NB + 1:NB + 1 + NB]
        sems = rest[NB + 1 + NB:]

        c = lax.axis_index("c")
        s = lax.axis_index("s")
        wid = s * NC + c

        _zero_fill(rows[0], C, D)
        rs = s * RPS

        @pl.loop(0, RPS // C)
        def _(t):
            pltpu.sync_copy(rows[0], aggr_sh.at[pl.ds(rs + t * C, C)])

        plsc.subcore_barrier()

        def g_start(b, j):
            pltpu.async_copy(x_hbm.at[idx_r.at[j]], rows[b], semg[b])

        def g_wait(b, j):
            pltpu.make_async_copy(x_hbm.at[idx_r.at[j]], rows[b], semg[b]).wait()

        def s_start(b, j):
            pltpu.async_copy(rows[b], aggr_sh.at[idx_c.at[j]], sems[b], add=True)

        def s_wait(b, j):
            pltpu.make_async_copy(rows[b], aggr_sh.at[idx_c.at[j]], sems[b]).wait()

        for off, nchunks in PHASES:
            pltpu.sync_copy(row_hbm.at[wid, pl.ds(off, IB)], idx_r)
            pltpu.sync_copy(col_hbm.at[wid, pl.ds(off, IB)], idx_c)

            FULL = nchunks // NB
            for b in range(NB):
                g_start(b, b)

            @pl.loop(0, FULL - 1)
            def _(g):
                j = NB * g
                for b in range(NB):
                    g_wait(b, j + b)
                    s_start(b, j + b)
                for b in range(NB):
                    s_wait(b, j + b)
                    g_start(b, j + NB + b)

            jl = NB * (FULL - 1)
            for b in range(NB):
                g_wait(b, jl + b)
                s_start(b, jl + b)
            for b in range(NB):
                s_wait(b, jl + b)

            for j in range(NB * FULL, nchunks):
                pltpu.async_copy(x_hbm.at[idx_r.at[j]], rows[0], semg[0]).wait()
                pltpu.sync_copy(rows[0], aggr_sh.at[idx_c.at[j]], add=True)

        plsc.subcore_barrier()
        pltpu.sync_copy(aggr_sh.at[pl.ds(rs, RPS)],
                        paggr_hbm.at[c, pl.ds(rs, RPS)])

    return k(x, row3, col3)


BR = 2048


def _combine_body(relu, p_ref, d_ref, x_ref, w_ref, b_ref, o_ref):
    d = d_ref[0, :, 0:1] + d_ref[1, :, 0:1]
    inv = 1.0 / jnp.maximum(d, 1.0)
    a = (p_ref[0] + p_ref[1]) * inv + x_ref[...]
    y = jnp.dot(a, w_ref[...], preferred_element_type=jnp.float32) + b_ref[...]
    o_ref[...] = jnp.maximum(y, 0.0) if relu else y


def _combine(p, degp, x, w, b, relu):
    return pl.pallas_call(
        functools.partial(_combine_body, relu),
        grid=(NPAD // BR,),
        in_specs=[
            pl.BlockSpec((NC, BR, D), lambda i: (0, i, 0)),
            pl.BlockSpec((NC, BR, 16), lambda i: (0, i, 0)),
            pl.BlockSpec((BR, D), lambda i: (i, 0)),
            pl.BlockSpec((D, D), lambda i: (0, 0)),
            pl.BlockSpec((1, D), lambda i: (0, 0)),
        ],
        out_specs=pl.BlockSpec((BR, D), lambda i: (i, 0)),
        out_shape=jax.ShapeDtypeStruct((NPAD, D), jnp.float32),
    )(p, degp, x, w, b.reshape(1, D))


def kernel(x, edge_index, W0, b0, W1, b1, W2, b2):
    ei = edge_index.astype(jnp.int32)
    row = ei[0]
    col = ei[1]
    xp = jnp.pad(x, ((0, NPAD - N), (0, 0)))

    degp = _sc_deg(row)
    row3 = row.reshape(NW, ICH, C)
    col3 = col.reshape(NW, ICH, C)
    p1 = _sc_aggr(xp, row3, col3)
    h1 = _combine(p1, degp, xp, W0, b0, relu=True)
    p2 = _sc_aggr(h1, row3, col3)
    h2 = _combine(p2, degp, h1, W1, b1, relu=True)
    p3 = _sc_aggr(h2, row3, col3)
    out = _combine(p3, degp, h2, W2, b2, relu=False)
    return out[:N]


# speedup vs baseline: 11.6201x; 1.0026x over previous
import functools

import jax
import jax.numpy as jnp
from jax import lax
from jax.experimental import pallas as pl
from jax.experimental.pallas import tpu as pltpu
from jax.experimental.pallas import tpu_sc as plsc

N = 10000
NPAD = 10240
E = 320000
D = 128

NC = 2
NS = 16
NW = NC * NS
EPW = E // NW
C = 80
CD = 1000
ITERS = EPW // C
RPS = NPAD // NS
ZR = 32

_mesh = plsc.VectorSubcoreMesh(
    core_axis_name="c", subcore_axis_name="s", num_cores=NC, num_subcores=NS
)


def _zero_fill(buf, rows, cols):
    zv = jnp.zeros((16,), jnp.float32)

    @pl.loop(0, rows)
    def _(r):
        @pl.loop(0, cols, step=16)
        def _(j):
            buf.at[r, pl.ds(j, 16)][...] = zv


def _sc_deg(row, width=16, lin=True):
    @functools.partial(
        pl.kernel,
        out_type=jax.ShapeDtypeStruct((NC, NPAD, width), jnp.float32),
        mesh=_mesh,
        scratch_types=[
            pltpu.VMEM((CD,), jnp.int32),
            pltpu.VMEM((CD, width), jnp.float32),
            pltpu.VMEM((ZR, width), jnp.float32),
            pltpu.VMEM_SHARED((NPAD, width), jnp.float32),
            pltpu.SemaphoreType.DMA,
        ],
        compiler_params=pltpu.CompilerParams(use_tc_tiling_on_sc=not lin),
    )
    def k(row_hbm, pdeg_hbm, idx_r, ones_v, zdeg, deg_sh, sem):
        c = lax.axis_index("c")
        s = lax.axis_index("s")
        wid = s * NC + c

        _zero_fill(zdeg, ZR, width)
        ov = jnp.ones((16,), jnp.float32)

        @pl.loop(0, CD)
        def _(r):
            @pl.loop(0, width, step=16)
            def _(j):
                ones_v.at[r, pl.ds(j, 16)][...] = ov

        rs = s * RPS

        @pl.loop(0, RPS // ZR)
        def _(t):
            pltpu.sync_copy(zdeg, deg_sh.at[pl.ds(rs + t * ZR, ZR)])

        plsc.subcore_barrier()

        base = wid * EPW

        @pl.loop(0, EPW // CD)
        def _(i):
            off = base + i * CD
            pltpu.sync_copy(row_hbm.at[pl.ds(off, CD)], idx_r)
            pltpu.sync_copy(ones_v, deg_sh.at[idx_r], add=True)

        plsc.subcore_barrier()
        pltpu.sync_copy(deg_sh.at[pl.ds(rs, RPS)],
                        pdeg_hbm.at[c, pl.ds(rs, RPS)])

    return k(row)


ICH = EPW // C
NB = 3
IB = 63
PHASES = ((0, 62), (62, 63))


def _sc_aggr(x, row3, col3):
    @functools.partial(
        pl.kernel,
        out_type=jax.ShapeDtypeStruct((NC, NPAD, D), jnp.float32),
        mesh=_mesh,
        scratch_types=[
            pltpu.VMEM((IB, C), jnp.int32),
            pltpu.VMEM((IB, C), jnp.int32),
        ] + [pltpu.VMEM((C, D), jnp.float32)] * NB + [
            pltpu.VMEM_SHARED((NPAD, D), jnp.float32),
        ] + [pltpu.SemaphoreType.DMA] * (2 * NB),
        compiler_params=pltpu.CompilerParams(use_tc_tiling_on_sc=False),
    )
    def k(x_hbm, row_hbm, col_hbm, paggr_hbm, idx_r, idx_c, *rest):
        rows = rest[:NB]
        aggr_sh = rest[NB]
        semg = rest[NB + 1:NB + 1 + NB]
        sems = rest[NB + 1 + NB:]

        c = lax.axis_index("c")
        s = lax.axis_index("s")
        wid = s * NC + c

        _zero_fill(rows[0], C, D)
        rs = s * RPS

        @pl.loop(0, RPS // C)
        def _(t):
            pltpu.sync_copy(rows[0], aggr_sh.at[pl.ds(rs + t * C, C)])

        plsc.subcore_barrier()

        def g_start(b, j):
            pltpu.async_copy(x_hbm.at[idx_r.at[j]], rows[b], semg[b])

        def g_wait(b, j):
            pltpu.make_async_copy(x_hbm.at[idx_r.at[j]], rows[b], semg[b]).wait()

        def s_start(b, j):
            pltpu.async_copy(rows[b], aggr_sh.at[idx_c.at[j]], sems[b], add=True)

        def s_wait(b, j):
            pltpu.make_async_copy(rows[b], aggr_sh.at[idx_c.at[j]], sems[b]).wait()

        for off, nchunks in PHASES:
            pltpu.sync_copy(row_hbm.at[wid, pl.ds(off, IB)], idx_r)
            pltpu.sync_copy(col_hbm.at[wid, pl.ds(off, IB)], idx_c)

            FULL = nchunks // NB
            for b in range(NB):
                g_start(b, b)

            @pl.loop(0, FULL - 1)
            def _(g):
                j = NB * g
                for b in range(NB):
                    g_wait(b, j + b)
                    s_start(b, j + b)
                for b in range(NB):
                    s_wait(b, j + b)
                    g_start(b, j + NB + b)

            jl = NB * (FULL - 1)
            for b in range(NB):
                g_wait(b, jl + b)
                s_start(b, jl + b)
            for b in range(NB):
                s_wait(b, jl + b)

            for j in range(NB * FULL, nchunks):
                pltpu.async_copy(x_hbm.at[idx_r.at[j]], rows[0], semg[0]).wait()
                pltpu.sync_copy(rows[0], aggr_sh.at[idx_c.at[j]], add=True)

        plsc.subcore_barrier()
        pltpu.sync_copy(aggr_sh.at[pl.ds(rs, RPS)],
                        paggr_hbm.at[c, pl.ds(rs, RPS)])

    return k(x, row3, col3)


BR = 2048


def _mm_body(x_ref, w_ref, o_ref):
    o_ref[...] = jnp.dot(x_ref[...], w_ref[...],
                         preferred_element_type=jnp.float32)


def _mm(x, w):
    return pl.pallas_call(
        _mm_body,
        grid=(NPAD // BR,),
        in_specs=[
            pl.BlockSpec((BR, D), lambda i: (i, 0)),
            pl.BlockSpec((D, D), lambda i: (0, 0)),
        ],
        out_specs=pl.BlockSpec((BR, D), lambda i: (i, 0)),
        out_shape=jax.ShapeDtypeStruct((NPAD, D), jnp.float32),
    )(x, w)


def _combine_body(relu, matmul, p_ref, d_ref, u_ref, b_ref, w_ref, o_ref):
    d = d_ref[0, :, 0:1] + d_ref[1, :, 0:1]
    inv = 1.0 / jnp.maximum(d, 1.0)
    y = (p_ref[0] + p_ref[1]) * inv + u_ref[...] + b_ref[...]
    if relu:
        y = jnp.maximum(y, 0.0)
    if matmul:
        y = jnp.dot(y, w_ref[...], preferred_element_type=jnp.float32)
    o_ref[...] = y


def _combine(p, degp, u, b, w, relu, matmul):
    return pl.pallas_call(
        functools.partial(_combine_body, relu, matmul),
        grid=(NPAD // BR,),
        in_specs=[
            pl.BlockSpec((NC, BR, D), lambda i: (0, i, 0)),
            pl.BlockSpec((NC, BR, 16), lambda i: (0, i, 0)),
            pl.BlockSpec((BR, D), lambda i: (i, 0)),
            pl.BlockSpec((1, D), lambda i: (0, 0)),
            pl.BlockSpec((D, D), lambda i: (0, 0)),
        ],
        out_specs=pl.BlockSpec((BR, D), lambda i: (i, 0)),
        out_shape=jax.ShapeDtypeStruct((NPAD, D), jnp.float32),
    )(p, degp, u, b.reshape(1, D), w)


def kernel(x, edge_index, W0, b0, W1, b1, W2, b2):
    ei = edge_index.astype(jnp.int32)
    row = ei[0]
    col = ei[1]
    xp = jnp.pad(x, ((0, NPAD - N), (0, 0)))
    row3 = row.reshape(NW, ICH, C)
    col3 = col.reshape(NW, ICH, C)

    degp = _sc_deg(row)
    u0 = _mm(xp, W0)
    p1 = _sc_aggr(u0, row3, col3)
    u1 = _combine(p1, degp, u0, b0, W1, relu=True, matmul=True)
    p2 = _sc_aggr(u1, row3, col3)
    u2 = _combine(p2, degp, u1, b1, W2, relu=True, matmul=True)
    p3 = _sc_aggr(u2, row3, col3)
    out = _combine(p3, degp, u2, b2, W2, relu=False, matmul=False)
    return out[:N]


# speedup vs baseline: 11.6313x; 1.0010x over previous
import functools

import jax
import jax.numpy as jnp
from jax import lax
from jax.experimental import pallas as pl
from jax.experimental.pallas import tpu as pltpu
from jax.experimental.pallas import tpu_sc as plsc

N = 10000
NPAD = 10240
E = 320000
D = 128

NC = 2
NS = 16
NW = NC * NS
EPW = E // NW
C = 80
CD = 1000
ITERS = EPW // C
RPS = NPAD // NS
ZR = 32

_mesh = plsc.VectorSubcoreMesh(
    core_axis_name="c", subcore_axis_name="s", num_cores=NC, num_subcores=NS
)


def _zero_fill(buf, rows, cols):
    zv = jnp.zeros((16,), jnp.float32)

    @pl.loop(0, rows)
    def _(r):
        @pl.loop(0, cols, step=16)
        def _(j):
            buf.at[r, pl.ds(j, 16)][...] = zv


def _sc_deg(row, width=16, lin=True):
    @functools.partial(
        pl.kernel,
        out_type=jax.ShapeDtypeStruct((NC, NPAD, width), jnp.float32),
        mesh=_mesh,
        scratch_types=[
            pltpu.VMEM((CD,), jnp.int32),
            pltpu.VMEM((CD, width), jnp.float32),
            pltpu.VMEM((ZR, width), jnp.float32),
            pltpu.VMEM_SHARED((NPAD, width), jnp.float32),
            pltpu.SemaphoreType.DMA,
        ],
        compiler_params=pltpu.CompilerParams(use_tc_tiling_on_sc=not lin),
    )
    def k(row_hbm, pdeg_hbm, idx_r, ones_v, zdeg, deg_sh, sem):
        c = lax.axis_index("c")
        s = lax.axis_index("s")
        wid = s * NC + c

        _zero_fill(zdeg, ZR, width)
        ov = jnp.ones((16,), jnp.float32)

        @pl.loop(0, CD)
        def _(r):
            @pl.loop(0, width, step=16)
            def _(j):
                ones_v.at[r, pl.ds(j, 16)][...] = ov

        rs = s * RPS

        @pl.loop(0, RPS // ZR)
        def _(t):
            pltpu.sync_copy(zdeg, deg_sh.at[pl.ds(rs + t * ZR, ZR)])

        plsc.subcore_barrier()

        base = wid * EPW

        @pl.loop(0, EPW // CD)
        def _(i):
            off = base + i * CD
            pltpu.sync_copy(row_hbm.at[pl.ds(off, CD)], idx_r)
            pltpu.sync_copy(ones_v, deg_sh.at[idx_r], add=True)

        plsc.subcore_barrier()
        pltpu.sync_copy(deg_sh.at[pl.ds(rs, RPS)],
                        pdeg_hbm.at[c, pl.ds(rs, RPS)])

    return k(row)


ICH = EPW // C
NB = 3
IB = 63
PHASES = ((0, 62), (62, 63))


def _sc_aggr(x, row3, col3):
    @functools.partial(
        pl.kernel,
        out_type=jax.ShapeDtypeStruct((NC, NPAD, D), jnp.float32),
        mesh=_mesh,
        scratch_types=[
            pltpu.VMEM((IB, C), jnp.int32),
            pltpu.VMEM((IB, C), jnp.int32),
        ] + [pltpu.VMEM((C, D), jnp.float32)] * NB + [
            pltpu.VMEM_SHARED((NPAD, D), jnp.float32),
        ] + [pltpu.SemaphoreType.DMA] * (2 * NB),
        compiler_params=pltpu.CompilerParams(use_tc_tiling_on_sc=False),
    )
    def k(x_hbm, row_hbm, col_hbm, paggr_hbm, idx_r, idx_c, *rest):
        rows = rest[:NB]
        aggr_sh = rest[NB]
        semg = rest[NB + 1:NB + 1 + NB]
        sems = rest[NB + 1 + NB:]

        c = lax.axis_index("c")
        s = lax.axis_index("s")
        wid = s * NC + c

        _zero_fill(rows[0], C, D)
        rs = s * RPS

        @pl.loop(0, RPS // C)
        def _(t):
            pltpu.sync_copy(rows[0], aggr_sh.at[pl.ds(rs + t * C, C)])

        plsc.subcore_barrier()

        def g_start(b, j):
            pltpu.async_copy(x_hbm.at[idx_r.at[j]], rows[b], semg[b])

        def g_wait(b, j):
            pltpu.make_async_copy(x_hbm.at[idx_r.at[j]], rows[b], semg[b]).wait()

        def s_start(b, j):
            pltpu.async_copy(rows[b], aggr_sh.at[idx_c.at[j]], sems[b], add=True)

        def s_wait(b, j):
            pltpu.make_async_copy(rows[b], aggr_sh.at[idx_c.at[j]], sems[b]).wait()

        for off, nchunks in PHASES:
            pltpu.sync_copy(row_hbm.at[wid, pl.ds(off, IB)], idx_r)
            pltpu.sync_copy(col_hbm.at[wid, pl.ds(off, IB)], idx_c)

            FULL = nchunks // NB
            for b in range(NB):
                g_start(b, b)

            @pl.loop(0, FULL - 1)
            def _(g):
                j = NB * g
                for b in range(NB):
                    g_wait(b, j + b)
                    s_start(b, j + b)
                for b in range(NB):
                    s_wait(b, j + b)
                    g_start(b, j + NB + b)

            jl = NB * (FULL - 1)
            for b in range(NB):
                g_wait(b, jl + b)
                s_start(b, jl + b)
            for b in range(NB):
                s_wait(b, jl + b)

            for j in range(NB * FULL, nchunks):
                pltpu.async_copy(x_hbm.at[idx_r.at[j]], rows[0], semg[0]).wait()
                pltpu.sync_copy(rows[0], aggr_sh.at[idx_c.at[j]], add=True)

        plsc.subcore_barrier()
        pltpu.sync_copy(aggr_sh.at[pl.ds(rs, RPS)],
                        paggr_hbm.at[c, pl.ds(rs, RPS)])

    return k(x, row3, col3)


BR = 2048


def _mm_body(x_ref, w_ref, o_ref):
    o_ref[...] = jnp.dot(x_ref[...], w_ref[...],
                         preferred_element_type=jnp.float32)


_tc_params = pltpu.CompilerParams(dimension_semantics=("parallel",))


def _mm(x, w):
    return pl.pallas_call(
        _mm_body,
        grid=(NPAD // BR,),
        compiler_params=_tc_params,
        in_specs=[
            pl.BlockSpec((BR, D), lambda i: (i, 0)),
            pl.BlockSpec((D, D), lambda i: (0, 0)),
        ],
        out_specs=pl.BlockSpec((BR, D), lambda i: (i, 0)),
        out_shape=jax.ShapeDtypeStruct((NPAD, D), jnp.float32),
    )(x, w)


def _combine_body(relu, matmul, p_ref, d_ref, u_ref, b_ref, w_ref, o_ref):
    d = d_ref[0, :, 0:1] + d_ref[1, :, 0:1]
    inv = 1.0 / jnp.maximum(d, 1.0)
    y = (p_ref[0] + p_ref[1]) * inv + u_ref[...] + b_ref[...]
    if relu:
        y = jnp.maximum(y, 0.0)
    if matmul:
        y = jnp.dot(y, w_ref[...], preferred_element_type=jnp.float32)
    o_ref[...] = y


def _combine(p, degp, u, b, w, relu, matmul):
    return pl.pallas_call(
        functools.partial(_combine_body, relu, matmul),
        grid=(NPAD // BR,),
        compiler_params=_tc_params,
        in_specs=[
            pl.BlockSpec((NC, BR, D), lambda i: (0, i, 0)),
            pl.BlockSpec((NC, BR, 16), lambda i: (0, i, 0)),
            pl.BlockSpec((BR, D), lambda i: (i, 0)),
            pl.BlockSpec((1, D), lambda i: (0, 0)),
            pl.BlockSpec((D, D), lambda i: (0, 0)),
        ],
        out_specs=pl.BlockSpec((BR, D), lambda i: (i, 0)),
        out_shape=jax.ShapeDtypeStruct((NPAD, D), jnp.float32),
    )(p, degp, u, b.reshape(1, D), w)


def kernel(x, edge_index, W0, b0, W1, b1, W2, b2):
    ei = edge_index.astype(jnp.int32)
    row = ei[0]
    col = ei[1]
    xp = jnp.pad(x, ((0, NPAD - N), (0, 0)))
    row3 = row.reshape(NW, ICH, C)
    col3 = col.reshape(NW, ICH, C)

    degp = _sc_deg(row)
    u0 = _mm(xp, W0)
    p1 = _sc_aggr(u0, row3, col3)
    u1 = _combine(p1, degp, u0, b0, W1, relu=True, matmul=True)
    p2 = _sc_aggr(u1, row3, col3)
    u2 = _combine(p2, degp, u1, b1, W2, relu=True, matmul=True)
    p3 = _sc_aggr(u2, row3, col3)
    out = _combine(p3, degp, u2, b2, W2, relu=False, matmul=False)
    return out[:N]


# speedup vs baseline: 11.7567x; 1.0108x over previous
import functools

import jax
import jax.numpy as jnp
from jax import lax
from jax.experimental import pallas as pl
from jax.experimental.pallas import tpu as pltpu
from jax.experimental.pallas import tpu_sc as plsc

N = 10000
NPAD = 10240
E = 320000
D = 128

NC = 2
NS = 16
NW = NC * NS
EPW = E // NW
C = 80
CD = 1000
ITERS = EPW // C
RPS = NPAD // NS
ZR = 32

_mesh = plsc.VectorSubcoreMesh(
    core_axis_name="c", subcore_axis_name="s", num_cores=NC, num_subcores=NS
)


def _zero_fill(buf, rows, cols):
    zv = jnp.zeros((16,), jnp.float32)

    @pl.loop(0, rows)
    def _(r):
        @pl.loop(0, cols, step=16)
        def _(j):
            buf.at[r, pl.ds(j, 16)][...] = zv


def _sc_deg(row, width=16, lin=True):
    @functools.partial(
        pl.kernel,
        out_type=jax.ShapeDtypeStruct((NC, NPAD, width), jnp.float32),
        mesh=_mesh,
        scratch_types=[
            pltpu.VMEM((CD,), jnp.int32),
            pltpu.VMEM((CD, width), jnp.float32),
            pltpu.VMEM((ZR, width), jnp.float32),
            pltpu.VMEM_SHARED((NPAD, width), jnp.float32),
            pltpu.SemaphoreType.DMA,
        ],
        compiler_params=pltpu.CompilerParams(use_tc_tiling_on_sc=not lin),
    )
    def k(row_hbm, pdeg_hbm, idx_r, ones_v, zdeg, deg_sh, sem):
        c = lax.axis_index("c")
        s = lax.axis_index("s")
        wid = s * NC + c

        _zero_fill(zdeg, ZR, width)
        ov = jnp.ones((16,), jnp.float32)

        @pl.loop(0, CD)
        def _(r):
            @pl.loop(0, width, step=16)
            def _(j):
                ones_v.at[r, pl.ds(j, 16)][...] = ov

        rs = s * RPS

        @pl.loop(0, RPS // ZR)
        def _(t):
            pltpu.sync_copy(zdeg, deg_sh.at[pl.ds(rs + t * ZR, ZR)])

        plsc.subcore_barrier()

        base = wid * EPW

        @pl.loop(0, EPW // CD)
        def _(i):
            off = base + i * CD
            pltpu.sync_copy(row_hbm.at[pl.ds(off, CD)], idx_r)
            pltpu.sync_copy(ones_v, deg_sh.at[idx_r], add=True)

        plsc.subcore_barrier()
        pltpu.sync_copy(deg_sh.at[pl.ds(rs, RPS)],
                        pdeg_hbm.at[c, pl.ds(rs, RPS)])

    return k(row)


ICH = EPW // C
NB = 3
IB = 63
PHASES = ((0, 62), (62, 63))
IB_D = 32
PHASES_D = ((0, 32), (32, 32), (64, 32), (96, 29))


def _sc_aggr(x, row3, col3, do_deg):
    ib = IB_D if do_deg else IB
    phases = PHASES_D if do_deg else PHASES

    out_type = jax.ShapeDtypeStruct((NC, NPAD, D), jnp.float32)
    scratch = [
        pltpu.VMEM((ib, C), jnp.int32),
        pltpu.VMEM((ib, C), jnp.int32),
    ] + [pltpu.VMEM((C, D), jnp.float32)] * NB + [
        pltpu.VMEM_SHARED((NPAD, D), jnp.float32),
    ] + [pltpu.SemaphoreType.DMA] * (2 * NB)
    if do_deg:
        out_type = (out_type, jax.ShapeDtypeStruct((NC, NPAD, 16), jnp.float32))
        scratch += [
            pltpu.VMEM((C, 16), jnp.float32),
            pltpu.VMEM_SHARED((NPAD, 16), jnp.float32),
            pltpu.SemaphoreType.DMA,
        ]

    @functools.partial(
        pl.kernel,
        out_type=out_type,
        mesh=_mesh,
        scratch_types=scratch,
        compiler_params=pltpu.CompilerParams(use_tc_tiling_on_sc=False),
    )
    def k(x_hbm, row_hbm, col_hbm, *rest):
        if do_deg:
            (paggr_hbm, pdeg_hbm, idx_r, idx_c, *rest2) = rest
            ones_v, deg_sh, semd = rest2[3 * NB + 1:]
        else:
            (paggr_hbm, idx_r, idx_c, *rest2) = rest
        rows = rest2[:NB]
        aggr_sh = rest2[NB]
        semg = rest2[NB + 1:NB + 1 + NB]
        sems = rest2[NB + 1 + NB:NB + 1 + 2 * NB]

        c = lax.axis_index("c")
        s = lax.axis_index("s")
        wid = s * NC + c
        rs = s * RPS

        _zero_fill(rows[0], C, D)

        @pl.loop(0, RPS // C)
        def _(t):
            pltpu.sync_copy(rows[0], aggr_sh.at[pl.ds(rs + t * C, C)])

        if do_deg:
            _zero_fill(ones_v, C, 16)

            @pl.loop(0, RPS // C)
            def _(t):
                pltpu.sync_copy(ones_v, deg_sh.at[pl.ds(rs + t * C, C)])

            ov = jnp.ones((16,), jnp.float32)

            @pl.loop(0, C)
            def _(r):
                ones_v.at[r][...] = ov

        plsc.subcore_barrier()

        def g_start(b, j):
            pltpu.async_copy(x_hbm.at[idx_r.at[j]], rows[b], semg[b])

        def g_wait(b, j):
            pltpu.make_async_copy(x_hbm.at[idx_r.at[j]], rows[b], semg[b]).wait()

        def s_start(b, j):
            pltpu.async_copy(rows[b], aggr_sh.at[idx_c.at[j]], sems[b], add=True)

        def s_wait(b, j):
            pltpu.make_async_copy(rows[b], aggr_sh.at[idx_c.at[j]], sems[b]).wait()

        def d_start(j):
            if do_deg:
                pltpu.async_copy(ones_v, deg_sh.at[idx_r.at[j]], semd, add=True)

        def d_wait(j):
            if do_deg:
                pltpu.make_async_copy(ones_v, deg_sh.at[idx_r.at[j]], semd).wait()

        for off, nchunks in phases:
            lo = min(off, ICH - ib)
            lb = off - lo
            pltpu.sync_copy(row_hbm.at[wid, pl.ds(lo, ib)], idx_r)
            pltpu.sync_copy(col_hbm.at[wid, pl.ds(lo, ib)], idx_c)

            FULL = nchunks // NB
            for b in range(NB):
                g_start(b, lb + b)

            @pl.loop(0, FULL - 1)
            def _(g):
                j = lb + NB * g
                for b in range(NB):
                    g_wait(b, j + b)
                    s_start(b, j + b)
                    d_start(j + b)
                for b in range(NB):
                    s_wait(b, j + b)
                    d_wait(j + b)
                    g_start(b, j + NB + b)

            jl = lb + NB * (FULL - 1)
            for b in range(NB):
                g_wait(b, jl + b)
                s_start(b, jl + b)
                d_start(jl + b)
            for b in range(NB):
                s_wait(b, jl + b)
                d_wait(jl + b)

            for j in range(lb + NB * FULL, lb + nchunks):
                pltpu.async_copy(x_hbm.at[idx_r.at[j]], rows[0], semg[0]).wait()
                pltpu.sync_copy(rows[0], aggr_sh.at[idx_c.at[j]], add=True)
                d_start(j)
                d_wait(j)

        plsc.subcore_barrier()
        pltpu.sync_copy(aggr_sh.at[pl.ds(rs, RPS)],
                        paggr_hbm.at[c, pl.ds(rs, RPS)])
        if do_deg:
            pltpu.sync_copy(deg_sh.at[pl.ds(rs, RPS)],
                            pdeg_hbm.at[c, pl.ds(rs, RPS)])

    return k(x, row3, col3)


BR = 2048

_tc_params = pltpu.CompilerParams(dimension_semantics=("parallel",))


def _combine_body(relu, p_ref, d_ref, x_ref, w_ref, b_ref, o_ref):
    d = d_ref[0, :, 0:1] + d_ref[1, :, 0:1]
    inv = 1.0 / jnp.maximum(d, 1.0)
    a = (p_ref[0] + p_ref[1]) * inv + x_ref[...]
    y = jnp.dot(a, w_ref[...], preferred_element_type=jnp.float32) + b_ref[...]
    o_ref[...] = jnp.maximum(y, 0.0) if relu else y


def _combine(p, degp, x, w, b, relu):
    return pl.pallas_call(
        functools.partial(_combine_body, relu),
        grid=(NPAD // BR,),
        compiler_params=_tc_params,
        in_specs=[
            pl.BlockSpec((NC, BR, D), lambda i: (0, i, 0)),
            pl.BlockSpec((NC, BR, 16), lambda i: (0, i, 0)),
            pl.BlockSpec((BR, D), lambda i: (i, 0)),
            pl.BlockSpec((D, D), lambda i: (0, 0)),
            pl.BlockSpec((1, D), lambda i: (0, 0)),
        ],
        out_specs=pl.BlockSpec((BR, D), lambda i: (i, 0)),
        out_shape=jax.ShapeDtypeStruct((NPAD, D), jnp.float32),
    )(p, degp, x, w, b.reshape(1, D))


def kernel(x, edge_index, W0, b0, W1, b1, W2, b2):
    ei = edge_index.astype(jnp.int32)
    row = ei[0]
    col = ei[1]
    xp = jnp.pad(x, ((0, NPAD - N), (0, 0)))
    row3 = row.reshape(NW, ICH, C)
    col3 = col.reshape(NW, ICH, C)

    p1, degp = _sc_aggr(xp, row3, col3, do_deg=True)
    h1 = _combine(p1, degp, xp, W0, b0, relu=True)
    p2 = _sc_aggr(h1, row3, col3, do_deg=False)
    h2 = _combine(p2, degp, h1, W1, b1, relu=True)
    p3 = _sc_aggr(h2, row3, col3, do_deg=False)
    out = _combine(p3, degp, h2, W2, b2, relu=False)
    return out[:N]


# speedup vs baseline: 11.9290x; 1.0147x over previous
import functools

import jax
import jax.numpy as jnp
from jax import lax
from jax.experimental import pallas as pl
from jax.experimental.pallas import tpu as pltpu
from jax.experimental.pallas import tpu_sc as plsc

N = 10000
NPAD = 10240
E = 320000
D = 128

NC = 2
NS = 16
NW = NC * NS
EPW = E // NW
C = 80
CD = 1000
ITERS = EPW // C
RPS = NPAD // NS
ZR = 32

_mesh = plsc.VectorSubcoreMesh(
    core_axis_name="c", subcore_axis_name="s", num_cores=NC, num_subcores=NS
)


def _zero_fill(buf, rows, cols):
    zv = jnp.zeros((16,), jnp.float32)

    @pl.loop(0, rows)
    def _(r):
        @pl.loop(0, cols, step=16)
        def _(j):
            buf.at[r, pl.ds(j, 16)][...] = zv


def _sc_deg(row, width=16, lin=True):
    @functools.partial(
        pl.kernel,
        out_type=jax.ShapeDtypeStruct((NC, NPAD, width), jnp.float32),
        mesh=_mesh,
        scratch_types=[
            pltpu.VMEM((CD,), jnp.int32),
            pltpu.VMEM((CD, width), jnp.float32),
            pltpu.VMEM((ZR, width), jnp.float32),
            pltpu.VMEM_SHARED((NPAD, width), jnp.float32),
            pltpu.SemaphoreType.DMA,
        ],
        compiler_params=pltpu.CompilerParams(use_tc_tiling_on_sc=not lin),
    )
    def k(row_hbm, pdeg_hbm, idx_r, ones_v, zdeg, deg_sh, sem):
        c = lax.axis_index("c")
        s = lax.axis_index("s")
        wid = s * NC + c

        _zero_fill(zdeg, ZR, width)
        ov = jnp.ones((16,), jnp.float32)

        @pl.loop(0, CD)
        def _(r):
            @pl.loop(0, width, step=16)
            def _(j):
                ones_v.at[r, pl.ds(j, 16)][...] = ov

        rs = s * RPS

        @pl.loop(0, RPS // ZR)
        def _(t):
            pltpu.sync_copy(zdeg, deg_sh.at[pl.ds(rs + t * ZR, ZR)])

        plsc.subcore_barrier()

        base = wid * EPW

        @pl.loop(0, EPW // CD)
        def _(i):
            off = base + i * CD
            pltpu.sync_copy(row_hbm.at[pl.ds(off, CD)], idx_r)
            pltpu.sync_copy(ones_v, deg_sh.at[idx_r], add=True)

        plsc.subcore_barrier()
        pltpu.sync_copy(deg_sh.at[pl.ds(rs, RPS)],
                        pdeg_hbm.at[c, pl.ds(rs, RPS)])

    return k(row)


ICH = EPW // C
NB = 3
IB = 63
PHASES = ((0, 62), (62, 63))
IB_D = 32
PHASES_D = ((0, 32), (32, 32), (64, 32), (96, 29))


def _sc_aggr(x, row3, col3, do_deg):
    ib = IB_D if do_deg else IB
    phases = PHASES_D if do_deg else PHASES

    out_type = jax.ShapeDtypeStruct((NC, NPAD, D), jnp.float32)
    scratch = [
        pltpu.VMEM((ib, C), jnp.int32),
        pltpu.VMEM((ib, C), jnp.int32),
    ] + [pltpu.VMEM((C, D), jnp.float32)] * NB + [
        pltpu.VMEM_SHARED((NPAD, D), jnp.float32),
    ] + [pltpu.SemaphoreType.DMA] * (2 * NB)
    if do_deg:
        out_type = (out_type, jax.ShapeDtypeStruct((NC, NPAD, 16), jnp.float32))
        scratch += [
            pltpu.VMEM((C, 16), jnp.float32),
            pltpu.VMEM_SHARED((NPAD, 16), jnp.float32),
            pltpu.SemaphoreType.DMA,
        ]

    @functools.partial(
        pl.kernel,
        out_type=out_type,
        mesh=_mesh,
        scratch_types=scratch,
        compiler_params=pltpu.CompilerParams(use_tc_tiling_on_sc=False),
    )
    def k(x_hbm, row_hbm, col_hbm, *rest):
        if do_deg:
            (paggr_hbm, pdeg_hbm, idx_r, idx_c, *rest2) = rest
            ones_v, deg_sh, semd = rest2[3 * NB + 1:]
        else:
            (paggr_hbm, idx_r, idx_c, *rest2) = rest
        rows = rest2[:NB]
        aggr_sh = rest2[NB]
        semg = rest2[NB + 1:NB + 1 + NB]
        sems = rest2[NB + 1 + NB:NB + 1 + 2 * NB]

        c = lax.axis_index("c")
        s = lax.axis_index("s")
        wid = s * NC + c
        rs = s * RPS

        _zero_fill(rows[0], C, D)

        @pl.loop(0, RPS // C)
        def _(t):
            pltpu.sync_copy(rows[0], aggr_sh.at[pl.ds(rs + t * C, C)])

        if do_deg:
            _zero_fill(ones_v, C, 16)

            @pl.loop(0, RPS // C)
            def _(t):
                pltpu.sync_copy(ones_v, deg_sh.at[pl.ds(rs + t * C, C)])

            ov = jnp.ones((16,), jnp.float32)

            @pl.loop(0, C)
            def _(r):
                ones_v.at[r][...] = ov

        plsc.subcore_barrier()

        def g_start(b, j):
            pltpu.async_copy(x_hbm.at[idx_r.at[j]], rows[b], semg[b])

        def g_wait(b, j):
            pltpu.make_async_copy(x_hbm.at[idx_r.at[j]], rows[b], semg[b]).wait()

        def s_start(b, j):
            pltpu.async_copy(rows[b], aggr_sh.at[idx_c.at[j]], sems[b], add=True)

        def s_wait(b, j):
            pltpu.make_async_copy(rows[b], aggr_sh.at[idx_c.at[j]], sems[b]).wait()

        def d_start(j):
            if do_deg:
                pltpu.async_copy(ones_v, deg_sh.at[idx_r.at[j]], semd, add=True)

        def d_wait(j):
            if do_deg:
                pltpu.make_async_copy(ones_v, deg_sh.at[idx_r.at[j]], semd).wait()

        for off, nchunks in phases:
            lo = min(off, ICH - ib)
            lb = off - lo
            pltpu.sync_copy(row_hbm.at[wid, pl.ds(lo, ib)], idx_r)
            pltpu.sync_copy(col_hbm.at[wid, pl.ds(lo, ib)], idx_c)

            FULL = nchunks // NB
            for b in range(NB):
                g_start(b, lb + b)

            @pl.loop(0, FULL - 1)
            def _(g):
                j = lb + NB * g
                for b in range(NB):
                    g_wait(b, j + b)
                    s_start(b, j + b)
                    d_start(j + b)
                for b in range(NB):
                    s_wait(b, j + b)
                    d_wait(j + b)
                    g_start(b, j + NB + b)

            jl = lb + NB * (FULL - 1)
            for b in range(NB):
                g_wait(b, jl + b)
                s_start(b, jl + b)
                d_start(jl + b)
            for b in range(NB):
                s_wait(b, jl + b)
                d_wait(jl + b)

            for j in range(lb + NB * FULL, lb + nchunks):
                pltpu.async_copy(x_hbm.at[idx_r.at[j]], rows[0], semg[0]).wait()
                pltpu.sync_copy(rows[0], aggr_sh.at[idx_c.at[j]], add=True)
                d_start(j)
                d_wait(j)

        plsc.subcore_barrier()
        pltpu.sync_copy(aggr_sh.at[pl.ds(rs, RPS)],
                        paggr_hbm.at[c, pl.ds(rs, RPS)])
        if do_deg:
            pltpu.sync_copy(deg_sh.at[pl.ds(rs, RPS)],
                            pdeg_hbm.at[c, pl.ds(rs, RPS)])

    return k(x, row3, col3)


BR = 2000

_tc_params = pltpu.CompilerParams(dimension_semantics=("parallel",))


def _combine_body(relu, p_ref, d_ref, x_ref, w_ref, b_ref, o_ref):
    d = d_ref[0, :, 0:1] + d_ref[1, :, 0:1]
    inv = 1.0 / jnp.maximum(d, 1.0)
    a = (p_ref[0] + p_ref[1]) * inv + x_ref[...]
    y = jnp.dot(a, w_ref[...], preferred_element_type=jnp.float32) + b_ref[...]
    o_ref[...] = jnp.maximum(y, 0.0) if relu else y


def _combine(p, degp, x, w, b, relu):
    return pl.pallas_call(
        functools.partial(_combine_body, relu),
        grid=(N // BR,),
        compiler_params=_tc_params,
        in_specs=[
            pl.BlockSpec((NC, BR, D), lambda i: (0, i, 0)),
            pl.BlockSpec((NC, BR, 16), lambda i: (0, i, 0)),
            pl.BlockSpec((BR, D), lambda i: (i, 0)),
            pl.BlockSpec((D, D), lambda i: (0, 0)),
            pl.BlockSpec((1, D), lambda i: (0, 0)),
        ],
        out_specs=pl.BlockSpec((BR, D), lambda i: (i, 0)),
        out_shape=jax.ShapeDtypeStruct((N, D), jnp.float32),
    )(p, degp, x, w, b.reshape(1, D))


def kernel(x, edge_index, W0, b0, W1, b1, W2, b2):
    ei = edge_index.astype(jnp.int32)
    row = ei[0]
    col = ei[1]
    row3 = row.reshape(NW, ICH, C)
    col3 = col.reshape(NW, ICH, C)

    p1, degp = _sc_aggr(x, row3, col3, do_deg=True)
    h1 = _combine(p1, degp, x, W0, b0, relu=True)
    p2 = _sc_aggr(h1, row3, col3, do_deg=False)
    h2 = _combine(p2, degp, h1, W1, b1, relu=True)
    p3 = _sc_aggr(h2, row3, col3, do_deg=False)
    return _combine(p3, degp, h2, W2, b2, relu=False)


# speedup vs baseline: 12.0787x; 1.0126x over previous
import functools

import jax
import jax.numpy as jnp
from jax import lax
from jax.experimental import pallas as pl
from jax.experimental.pallas import tpu as pltpu
from jax.experimental.pallas import tpu_sc as plsc

N = 10000
NPAD = 10240
E = 320000
D = 128

NC = 2
NS = 16
NW = NC * NS
EPW = E // NW
C = 80
CD = 1000
ITERS = EPW // C
RPS = N // NS
ZFULL = RPS // C
ZREM = RPS - ZFULL * C
ZR = 32

_mesh = plsc.VectorSubcoreMesh(
    core_axis_name="c", subcore_axis_name="s", num_cores=NC, num_subcores=NS
)


def _zero_fill(buf, rows, cols):
    zv = jnp.zeros((16,), jnp.float32)

    @pl.loop(0, rows)
    def _(r):
        @pl.loop(0, cols, step=16)
        def _(j):
            buf.at[r, pl.ds(j, 16)][...] = zv


def _sc_deg(row, width=16, lin=True):
    @functools.partial(
        pl.kernel,
        out_type=jax.ShapeDtypeStruct((NC, NPAD, width), jnp.float32),
        mesh=_mesh,
        scratch_types=[
            pltpu.VMEM((CD,), jnp.int32),
            pltpu.VMEM((CD, width), jnp.float32),
            pltpu.VMEM((ZR, width), jnp.float32),
            pltpu.VMEM_SHARED((NPAD, width), jnp.float32),
            pltpu.SemaphoreType.DMA,
        ],
        compiler_params=pltpu.CompilerParams(use_tc_tiling_on_sc=not lin),
    )
    def k(row_hbm, pdeg_hbm, idx_r, ones_v, zdeg, deg_sh, sem):
        c = lax.axis_index("c")
        s = lax.axis_index("s")
        wid = s * NC + c

        _zero_fill(zdeg, ZR, width)
        ov = jnp.ones((16,), jnp.float32)

        @pl.loop(0, CD)
        def _(r):
            @pl.loop(0, width, step=16)
            def _(j):
                ones_v.at[r, pl.ds(j, 16)][...] = ov

        rs = s * RPS

        @pl.loop(0, RPS // ZR)
        def _(t):
            pltpu.sync_copy(zdeg, deg_sh.at[pl.ds(rs + t * ZR, ZR)])

        plsc.subcore_barrier()

        base = wid * EPW

        @pl.loop(0, EPW // CD)
        def _(i):
            off = base + i * CD
            pltpu.sync_copy(row_hbm.at[pl.ds(off, CD)], idx_r)
            pltpu.sync_copy(ones_v, deg_sh.at[idx_r], add=True)

        plsc.subcore_barrier()
        pltpu.sync_copy(deg_sh.at[pl.ds(rs, RPS)],
                        pdeg_hbm.at[c, pl.ds(rs, RPS)])

    return k(row)


ICH = EPW // C
NB = 3
IB = ICH
PHASES = ((0, ICH),)
IB_D = 32
PHASES_D = ((0, 32), (32, 32), (64, 32), (96, 29))


def _sc_aggr(x, row3, col3, do_deg):
    ib = IB_D if do_deg else IB
    phases = PHASES_D if do_deg else PHASES

    out_type = jax.ShapeDtypeStruct((NC, N, D), jnp.float32)
    scratch = [
        pltpu.VMEM((ib, C), jnp.int32),
        pltpu.VMEM((ib, C), jnp.int32),
    ] + [pltpu.VMEM((C, D), jnp.float32)] * NB + [
        pltpu.VMEM_SHARED((N, D), jnp.float32),
    ] + [pltpu.SemaphoreType.DMA] * (2 * NB)
    if do_deg:
        out_type = (out_type, jax.ShapeDtypeStruct((NC, N, 16), jnp.float32))
        scratch += [
            pltpu.VMEM((C, 16), jnp.float32),
            pltpu.VMEM_SHARED((N, 16), jnp.float32),
            pltpu.SemaphoreType.DMA,
        ]

    @functools.partial(
        pl.kernel,
        out_type=out_type,
        mesh=_mesh,
        scratch_types=scratch,
        compiler_params=pltpu.CompilerParams(use_tc_tiling_on_sc=False),
    )
    def k(x_hbm, row_hbm, col_hbm, *rest):
        if do_deg:
            (paggr_hbm, pdeg_hbm, idx_r, idx_c, *rest2) = rest
            ones_v, deg_sh, semd = rest2[3 * NB + 1:]
        else:
            (paggr_hbm, idx_r, idx_c, *rest2) = rest
        rows = rest2[:NB]
        aggr_sh = rest2[NB]
        semg = rest2[NB + 1:NB + 1 + NB]
        sems = rest2[NB + 1 + NB:NB + 1 + 2 * NB]

        c = lax.axis_index("c")
        s = lax.axis_index("s")
        wid = s * NC + c
        rs = s * RPS

        _zero_fill(rows[0], C, D)

        @pl.loop(0, ZFULL)
        def _(t):
            pltpu.sync_copy(rows[0], aggr_sh.at[pl.ds(rs + t * C, C)])

        pltpu.sync_copy(rows[0].at[pl.ds(0, ZREM)],
                        aggr_sh.at[pl.ds(rs + ZFULL * C, ZREM)])

        if do_deg:
            _zero_fill(ones_v, C, 16)

            @pl.loop(0, ZFULL)
            def _(t):
                pltpu.sync_copy(ones_v, deg_sh.at[pl.ds(rs + t * C, C)])

            pltpu.sync_copy(ones_v.at[pl.ds(0, ZREM)],
                            deg_sh.at[pl.ds(rs + ZFULL * C, ZREM)])

            ov = jnp.ones((16,), jnp.float32)

            @pl.loop(0, C)
            def _(r):
                ones_v.at[r][...] = ov

        plsc.subcore_barrier()

        def g_start(b, j):
            pltpu.async_copy(x_hbm.at[idx_r.at[j]], rows[b], semg[b])

        def g_wait(b, j):
            pltpu.make_async_copy(x_hbm.at[idx_r.at[j]], rows[b], semg[b]).wait()

        def s_start(b, j):
            pltpu.async_copy(rows[b], aggr_sh.at[idx_c.at[j]], sems[b], add=True)

        def s_wait(b, j):
            pltpu.make_async_copy(rows[b], aggr_sh.at[idx_c.at[j]], sems[b]).wait()

        def d_start(j):
            if do_deg:
                pltpu.async_copy(ones_v, deg_sh.at[idx_r.at[j]], semd, add=True)

        def d_wait(j):
            if do_deg:
                pltpu.make_async_copy(ones_v, deg_sh.at[idx_r.at[j]], semd).wait()

        for off, nchunks in phases:
            lo = min(off, ICH - ib)
            lb = off - lo
            pltpu.sync_copy(row_hbm.at[wid, pl.ds(lo, ib)], idx_r)
            pltpu.sync_copy(col_hbm.at[wid, pl.ds(lo, ib)], idx_c)

            FULL = nchunks // NB
            for b in range(NB):
                g_start(b, lb + b)

            @pl.loop(0, FULL - 1)
            def _(g):
                j = lb + NB * g
                for b in range(NB):
                    g_wait(b, j + b)
                    s_start(b, j + b)
                    d_start(j + b)
                for b in range(NB):
                    s_wait(b, j + b)
                    d_wait(j + b)
                    g_start(b, j + NB + b)

            jl = lb + NB * (FULL - 1)
            for b in range(NB):
                g_wait(b, jl + b)
                s_start(b, jl + b)
                d_start(jl + b)
            for b in range(NB):
                s_wait(b, jl + b)
                d_wait(jl + b)

            for j in range(lb + NB * FULL, lb + nchunks):
                pltpu.async_copy(x_hbm.at[idx_r.at[j]], rows[0], semg[0]).wait()
                pltpu.sync_copy(rows[0], aggr_sh.at[idx_c.at[j]], add=True)
                d_start(j)
                d_wait(j)

        plsc.subcore_barrier()
        pltpu.sync_copy(aggr_sh.at[pl.ds(rs, RPS)],
                        paggr_hbm.at[c, pl.ds(rs, RPS)])
        if do_deg:
            pltpu.sync_copy(deg_sh.at[pl.ds(rs, RPS)],
                            pdeg_hbm.at[c, pl.ds(rs, RPS)])

    return k(x, row3, col3)


BR = 2000

_tc_params = pltpu.CompilerParams(dimension_semantics=("parallel",))


def _combine_body(relu, p_ref, d_ref, x_ref, w_ref, b_ref, o_ref):
    d = d_ref[0, :, 0:1] + d_ref[1, :, 0:1]
    inv = 1.0 / jnp.maximum(d, 1.0)
    a = (p_ref[0] + p_ref[1]) * inv + x_ref[...]
    y = jnp.dot(a, w_ref[...], preferred_element_type=jnp.float32) + b_ref[...]
    o_ref[...] = jnp.maximum(y, 0.0) if relu else y


def _combine(p, degp, x, w, b, relu):
    return pl.pallas_call(
        functools.partial(_combine_body, relu),
        grid=(N // BR,),
        compiler_params=_tc_params,
        in_specs=[
            pl.BlockSpec((NC, BR, D), lambda i: (0, i, 0)),
            pl.BlockSpec((NC, BR, 16), lambda i: (0, i, 0)),
            pl.BlockSpec((BR, D), lambda i: (i, 0)),
            pl.BlockSpec((D, D), lambda i: (0, 0)),
            pl.BlockSpec((1, D), lambda i: (0, 0)),
        ],
        out_specs=pl.BlockSpec((BR, D), lambda i: (i, 0)),
        out_shape=jax.ShapeDtypeStruct((N, D), jnp.float32),
    )(p, degp, x, w, b.reshape(1, D))


def kernel(x, edge_index, W0, b0, W1, b1, W2, b2):
    ei = edge_index.astype(jnp.int32)
    row = ei[0]
    col = ei[1]
    row3 = row.reshape(NW, ICH, C)
    col3 = col.reshape(NW, ICH, C)

    p1, degp = _sc_aggr(x, row3, col3, do_deg=True)
    h1 = _combine(p1, degp, x, W0, b0, relu=True)
    p2 = _sc_aggr(h1, row3, col3, do_deg=False)
    h2 = _combine(p2, degp, h1, W1, b1, relu=True)
    p3 = _sc_aggr(h2, row3, col3, do_deg=False)
    return _combine(p3, degp, h2, W2, b2, relu=False)
